# Initial kernel scaffold; baseline (speedup 1.0000x reference)
import jax
import jax.numpy as jnp
from jax.experimental import pallas as pl


def kernel(x, edge_index, ln_gamma, ln_beta, W_l, b_l, W_r):
    raise NotImplementedError("write your pallas kernel here")


# speedup vs baseline: 6.0381x; 6.0381x over previous
import dataclasses
import functools

import jax
import jax.numpy as jnp
from jax import lax
from jax.experimental import pallas as pl
from jax.experimental.pallas import tpu as pltpu
from jax.experimental.pallas import tpu_sc as plsc

N = 10000
D = 128
E = 320000

NC = 2
NS = 16
L = 16
EPC = E // NC
EPT = EPC // NS
CH = 80
NCHUNK = EPT // CH
N_PAD = 10240
RPT = N_PAD // NS
NZB = RPT // 128
CROWS = 640


def _ln_relu_body(x_ref, g_ref, b_ref, h_ref):
    x = x_ref[...]
    m = jnp.mean(x, axis=1, keepdims=True)
    d = x - m
    v = jnp.mean(d * d, axis=1, keepdims=True)
    h = d * lax.rsqrt(v + 1e-5) * g_ref[...] + b_ref[...]
    h_ref[...] = jnp.maximum(h, 0.0)


def _ln_relu(x, gamma, beta):
    blk = 1000
    return pl.pallas_call(
        _ln_relu_body,
        grid=(N // blk,),
        in_specs=[
            pl.BlockSpec((blk, D), lambda i: (i, 0)),
            pl.BlockSpec((1, D), lambda i: (0, 0)),
            pl.BlockSpec((1, D), lambda i: (0, 0)),
        ],
        out_specs=pl.BlockSpec((blk, D), lambda i: (i, 0)),
        out_shape=jax.ShapeDtypeStruct((N, D), jnp.float32),
    )(x, gamma.reshape(1, D), beta.reshape(1, D))


_sc_mesh = plsc.VectorSubcoreMesh(core_axis_name="c", subcore_axis_name="s")

_sc_params = pltpu.CompilerParams(
    needs_layout_passes=False,
    use_tc_tiling_on_sc=False,
)


@functools.partial(
    pl.kernel,
    out_type=(
        jax.ShapeDtypeStruct((NC, N_PAD, D), jnp.float32),
        jax.ShapeDtypeStruct((NC, CROWS, L), jnp.float32),
    ),
    mesh=_sc_mesh,
    compiler_params=_sc_params,
    scratch_types=[
        pltpu.VMEM((CH,), jnp.int32),
        pltpu.VMEM((CH,), jnp.int32),
        pltpu.VMEM((CH, D), jnp.float32),
        pltpu.VMEM((CROWS, L), jnp.float32),
        pltpu.VMEM((5, 128), jnp.int32),
        pltpu.VMEM((128, D), jnp.float32),
        pltpu.VMEM((128, L), jnp.float32),
        pltpu.VMEM_SHARED((N_PAD, D), jnp.float32),
        pltpu.VMEM_SHARED((CROWS, L), jnp.float32),
    ],
)
def _sc_aggregate(h_hbm, src_hbm, dst_hbm, aggr_out, cnt_out,
                  src_v, dst_v, rows_v, cnt_l, rowid_v, zrow_v, zcnt_v,
                  aggr_sh, cnt_sh):
    c = lax.axis_index("c")
    s = lax.axis_index("s")

    @pl.loop(0, 128)
    def _(i):
        @pl.loop(0, D, step=L)
        def _(j):
            zrow_v[i, pl.ds(j, L)] = jnp.zeros((L,), jnp.float32)

    @pl.loop(0, 128)
    def _(i):
        zcnt_v[i, pl.ds(0, L)] = jnp.zeros((L,), jnp.float32)

    @pl.loop(0, CROWS)
    def _(i):
        cnt_l[i, pl.ds(0, L)] = jnp.zeros((L,), jnp.float32)

    for k in range(5):
        for j in range(8):
            rowid_v[k, pl.ds(j * L, L)] = (
                lax.iota(jnp.int32, L) + (k * 128 + j * L)
            )

    @pl.loop(0, NZB)
    def _(k):
        pltpu.sync_copy(zrow_v, aggr_sh.at[pl.ds(s * RPT + k * 128, 128)])

    @pl.when(s == 0)
    def _():
        @pl.loop(0, 5)
        def _(k):
            pltpu.sync_copy(zcnt_v, cnt_sh.at[pl.ds(k * 128, 128)])

    plsc.subcore_barrier()

    e0 = c * EPC + s * EPT

    @pl.loop(0, NCHUNK)
    def _(i):
        base = e0 + i * CH
        pltpu.sync_copy(src_hbm.at[pl.ds(base, CH)], src_v)
        pltpu.sync_copy(dst_hbm.at[pl.ds(base, CH)], dst_v)
        pltpu.sync_copy(h_hbm.at[src_v], rows_v)
        pltpu.sync_copy(rows_v, aggr_sh.at[dst_v], add=True)
        for j in range(CH // L):
            dv = dst_v[pl.ds(j * L, L)]
            row = lax.shift_right_logical(dv, 4)
            col = lax.bitwise_and(dv, 15)
            plsc.addupdate_scatter(cnt_l, [row, col],
                                   jnp.full((L,), 1.0, jnp.float32))

    plsc.subcore_barrier()

    for k in range(5):
        pltpu.sync_copy(cnt_l.at[pl.ds(k * 128, 128)],
                        cnt_sh.at[rowid_v.at[k]], add=True)

    plsc.subcore_barrier()

    @pl.loop(0, NZB)
    def _(k):
        r0 = s * RPT + k * 128
        pltpu.sync_copy(aggr_sh.at[pl.ds(r0, 128)],
                        aggr_out.at[c, pl.ds(r0, 128)])

    @pl.when(s == 0)
    def _():
        pltpu.sync_copy(cnt_sh, cnt_out.at[c])


def _combine_body(a0_ref, a1_ref, c0_ref, c1_ref, h_ref, wl_ref, bl_ref,
                  wr_ref, o_ref):
    aggr = a0_ref[...] + a1_ref[...]
    cnt = jnp.maximum(c0_ref[...] + c1_ref[...], 1.0)
    mean = aggr / cnt
    dims = (((1,), (1,)), ((), ()))
    o_ref[...] = (
        lax.dot_general(mean, wl_ref[...], dims,
                        preferred_element_type=jnp.float32)
        + bl_ref[...]
        + lax.dot_general(h_ref[...], wr_ref[...], dims,
                          preferred_element_type=jnp.float32)
    )


def _combine(a0, a1, c0, c1, h, W_l, b_l, W_r):
    blk = 1000
    return pl.pallas_call(
        _combine_body,
        grid=(N // blk,),
        in_specs=[
            pl.BlockSpec((blk, D), lambda i: (i, 0)),
            pl.BlockSpec((blk, D), lambda i: (i, 0)),
            pl.BlockSpec((blk, 1), lambda i: (i, 0)),
            pl.BlockSpec((blk, 1), lambda i: (i, 0)),
            pl.BlockSpec((blk, D), lambda i: (i, 0)),
            pl.BlockSpec((D, D), lambda i: (0, 0)),
            pl.BlockSpec((1, D), lambda i: (0, 0)),
            pl.BlockSpec((D, D), lambda i: (0, 0)),
        ],
        out_specs=pl.BlockSpec((blk, D), lambda i: (i, 0)),
        out_shape=jax.ShapeDtypeStruct((N, D), jnp.float32),
    )(a0, a1, c0, c1, h, W_l, b_l.reshape(1, D), W_r)


def kernel(x, edge_index, ln_gamma, ln_beta, W_l, b_l, W_r):
    ei = edge_index.astype(jnp.int32)
    src = ei[0]
    dst = ei[1]
    h = _ln_relu(x, ln_gamma, ln_beta)
    aggr_p, cnt_p = _sc_aggregate(h, src, dst)
    aggr_p = aggr_p[:, :N, :]
    cnt = cnt_p.reshape(NC, CROWS * L)[:, :N]
    out = _combine(aggr_p[0], aggr_p[1],
                   cnt[0].reshape(N, 1), cnt[1].reshape(N, 1),
                   h, W_l, b_l, W_r)
    return out


# speedup vs baseline: 10.3935x; 1.7213x over previous
import dataclasses
import functools

import jax
import jax.numpy as jnp
from jax import lax
from jax.experimental import pallas as pl
from jax.experimental.pallas import tpu as pltpu
from jax.experimental.pallas import tpu_sc as plsc

N = 10000
D = 128
E = 320000

NC = 2
NS = 16
L = 16
EPC = E // NC
EPT = EPC // NS
CH = 80
NCHUNK = EPT // CH
NBUF = 2
NGROUP = NCHUNK // NBUF
NRING = NGROUP * NBUF
N_PAD = 10112
RPT = N_PAD // NS
RBLOCKS = [(k, min(128, RPT - k)) for k in range(0, RPT, 128)]
CROWS = 640


def _ln_relu_body(x_ref, g_ref, b_ref, h_ref):
    x = x_ref[...]
    m = jnp.mean(x, axis=1, keepdims=True)
    d = x - m
    v = jnp.mean(d * d, axis=1, keepdims=True)
    h = d * lax.rsqrt(v + 1e-5) * g_ref[...] + b_ref[...]
    h_ref[...] = jnp.maximum(h, 0.0)


def _ln_relu(x, gamma, beta):
    blk = 1000
    return pl.pallas_call(
        _ln_relu_body,
        grid=(N // blk,),
        in_specs=[
            pl.BlockSpec((blk, D), lambda i: (i, 0)),
            pl.BlockSpec((1, D), lambda i: (0, 0)),
            pl.BlockSpec((1, D), lambda i: (0, 0)),
        ],
        out_specs=pl.BlockSpec((blk, D), lambda i: (i, 0)),
        out_shape=jax.ShapeDtypeStruct((N, D), jnp.float32),
    )(x, gamma.reshape(1, D), beta.reshape(1, D))


_sc_mesh = plsc.VectorSubcoreMesh(core_axis_name="c", subcore_axis_name="s")

_sc_params = pltpu.CompilerParams(
    needs_layout_passes=False,
    use_tc_tiling_on_sc=False,
)


@functools.partial(
    pl.kernel,
    out_type=(
        jax.ShapeDtypeStruct((NC, N_PAD, D), jnp.float32),
        jax.ShapeDtypeStruct((NC, CROWS, L), jnp.float32),
    ),
    mesh=_sc_mesh,
    compiler_params=_sc_params,
    scratch_types=[
        pltpu.VMEM((NCHUNK, CH), jnp.int32),
        pltpu.VMEM((CROWS, L), jnp.float32),
        pltpu.VMEM((5, 128), jnp.int32),
        pltpu.VMEM_SHARED((N_PAD, D), jnp.float32),
        pltpu.VMEM_SHARED((CROWS, L), jnp.float32),
    ] + [pltpu.VMEM((CH, D), jnp.float32)] * NBUF
      + [pltpu.VMEM((CH,), jnp.int32)] * NBUF
      + [pltpu.SemaphoreType.DMA] * (3 * NBUF),
)
def _sc_aggregate(h_hbm, src_hbm, dst_hbm, aggr_out, cnt_out,
                  dst_m, cnt_l, rowid_v,
                  aggr_sh, cnt_sh, *rest):
    rows = rest[:NBUF]
    srcs = rest[NBUF:2 * NBUF]
    gsem = rest[2 * NBUF:3 * NBUF]
    ssem = rest[3 * NBUF:4 * NBUF]
    isem = rest[4 * NBUF:]
    c = lax.axis_index("c")
    s = lax.axis_index("s")

    @pl.loop(0, CH)
    def _(i):
        @pl.loop(0, D, step=L)
        def _(j):
            rows[0][i, pl.ds(j, L)] = jnp.zeros((L,), jnp.float32)

    @pl.loop(0, CROWS)
    def _(i):
        cnt_l[i, pl.ds(0, L)] = jnp.zeros((L,), jnp.float32)

    for k in range(5):
        for j in range(8):
            rowid_v[k, pl.ds(j * L, L)] = (
                lax.iota(jnp.int32, L) + (k * 128 + j * L)
            )

    for k in range(0, RPT, CH):
        nb = min(CH, RPT - k)
        pltpu.sync_copy(rows[0].at[pl.ds(0, nb)],
                        aggr_sh.at[pl.ds(s * RPT + k, nb)])

    @pl.when(s == 0)
    def _():
        for k in range(0, CROWS, 128):
            pltpu.sync_copy(cnt_l.at[pl.ds(k, 128)],
                            cnt_sh.at[pl.ds(k, 128)])

    plsc.subcore_barrier()

    wid = c * NS + s
    e0 = wid * EPT

    pltpu.sync_copy(dst_hbm.at[wid], dst_m)

    def src_desc(i, b):
        return pltpu.make_async_copy(
            src_hbm.at[pl.ds(e0 + i * CH, CH)], srcs[b], isem[b])

    def gather_desc(b):
        return pltpu.make_async_copy(
            h_hbm.at[srcs[b].at[pl.ds(0, CH)]], rows[b], gsem[b])

    def scatter_desc(i, b):
        return pltpu.make_async_copy(
            rows[b], aggr_sh.at[dst_m.at[i]], ssem[b])

    def count_update(i):
        for j in range(CH // L):
            dv = dst_m[i, pl.ds(j * L, L)]
            row = lax.shift_right_logical(dv, 4)
            col = lax.bitwise_and(dv, 15)
            plsc.addupdate_scatter(cnt_l, [row, col],
                                   jnp.full((L,), 1.0, jnp.float32))

    for b in range(NBUF):
        src_desc(b, b).start()
    for b in range(NBUF):
        src_desc(b, b).wait()
        gather_desc(b).start()

    @pl.loop(0, NGROUP)
    def _(g):
        for b in range(NBUF):
            i = g * NBUF + b
            gather_desc(b).wait()

            @pl.when(g < NGROUP - 1)
            def _():
                src_desc(i + NBUF, b).start()

            scatter_desc(i, b).start(add=True)
            count_update(i)
        for b in range(NBUF):
            i = g * NBUF + b
            scatter_desc(i, b).wait()

            @pl.when(g < NGROUP - 1)
            def _():
                src_desc(i + NBUF, b).wait()
                gather_desc(b).start()

    for i in range(NRING, NCHUNK):
        pltpu.sync_copy(src_hbm.at[pl.ds(e0 + i * CH, CH)], srcs[0])
        pltpu.sync_copy(h_hbm.at[srcs[0].at[pl.ds(0, CH)]], rows[0])
        pltpu.sync_copy(rows[0], aggr_sh.at[dst_m.at[i]], add=True)
        count_update(i)

    plsc.subcore_barrier()

    for k in range(5):
        pltpu.sync_copy(cnt_l.at[pl.ds(k * 128, 128)],
                        cnt_sh.at[rowid_v.at[k]], add=True)

    plsc.subcore_barrier()

    for k, nb in RBLOCKS:
        r0 = s * RPT + k
        pltpu.sync_copy(aggr_sh.at[pl.ds(r0, nb)],
                        aggr_out.at[c, pl.ds(r0, nb)])

    @pl.when(s == 0)
    def _():
        pltpu.sync_copy(cnt_sh, cnt_out.at[c])


def _combine_body(a0_ref, a1_ref, c0_ref, c1_ref, h_ref, wl_ref, bl_ref,
                  wr_ref, o_ref):
    aggr = a0_ref[...] + a1_ref[...]
    cnt = jnp.maximum(c0_ref[...] + c1_ref[...], 1.0)
    mean = aggr / cnt
    dims = (((1,), (1,)), ((), ()))
    o_ref[...] = (
        lax.dot_general(mean, wl_ref[...], dims,
                        preferred_element_type=jnp.float32)
        + bl_ref[...]
        + lax.dot_general(h_ref[...], wr_ref[...], dims,
                          preferred_element_type=jnp.float32)
    )


def _combine(a0, a1, c0, c1, h, W_l, b_l, W_r):
    blk = 1000
    return pl.pallas_call(
        _combine_body,
        grid=(N // blk,),
        in_specs=[
            pl.BlockSpec((blk, D), lambda i: (i, 0)),
            pl.BlockSpec((blk, D), lambda i: (i, 0)),
            pl.BlockSpec((blk, 1), lambda i: (i, 0)),
            pl.BlockSpec((blk, 1), lambda i: (i, 0)),
            pl.BlockSpec((blk, D), lambda i: (i, 0)),
            pl.BlockSpec((D, D), lambda i: (0, 0)),
            pl.BlockSpec((1, D), lambda i: (0, 0)),
            pl.BlockSpec((D, D), lambda i: (0, 0)),
        ],
        out_specs=pl.BlockSpec((blk, D), lambda i: (i, 0)),
        out_shape=jax.ShapeDtypeStruct((N, D), jnp.float32),
    )(a0, a1, c0, c1, h, W_l, b_l.reshape(1, D), W_r)


def kernel(x, edge_index, ln_gamma, ln_beta, W_l, b_l, W_r):
    ei = edge_index.astype(jnp.int32)
    src = ei[0]
    dst = ei[1]
    h = _ln_relu(x, ln_gamma, ln_beta)
    aggr_p, cnt_p = _sc_aggregate(h, src, dst.reshape(NC * NS, NCHUNK, CH))
    aggr_p = aggr_p[:, :N, :]
    cnt = cnt_p.reshape(NC, CROWS * L)[:, :N]
    out = _combine(aggr_p[0], aggr_p[1],
                   cnt[0].reshape(N, 1), cnt[1].reshape(N, 1),
                   h, W_l, b_l, W_r)
    return out


# speedup vs baseline: 12.0102x; 1.1555x over previous
import dataclasses
import functools

import jax
import jax.numpy as jnp
from jax import lax
from jax.experimental import pallas as pl
from jax.experimental.pallas import tpu as pltpu
from jax.experimental.pallas import tpu_sc as plsc

N = 10000
D = 128
E = 320000

NC = 2
NS = 16
L = 16
EPC = E // NC
EPT = EPC // NS
CH = 80
NCHUNK = EPT // CH
NBUF = 3
NGROUP = NCHUNK // NBUF
NRING = NGROUP * NBUF
N_PAD = 10112
RPT = N_PAD // NS
RBLOCKS = [(k, min(128, RPT - k)) for k in range(0, RPT, 128)]
CROWS = 640


def _ln_relu_body(x_ref, g_ref, b_ref, h_ref):
    x = x_ref[...]
    m = jnp.mean(x, axis=1, keepdims=True)
    d = x - m
    v = jnp.mean(d * d, axis=1, keepdims=True)
    h = d * lax.rsqrt(v + 1e-5) * g_ref[...] + b_ref[...]
    h_ref[...] = jnp.maximum(h, 0.0)


def _ln_relu(x, gamma, beta):
    blk = 1000
    return pl.pallas_call(
        _ln_relu_body,
        grid=(N // blk,),
        in_specs=[
            pl.BlockSpec((blk, D), lambda i: (i, 0)),
            pl.BlockSpec((1, D), lambda i: (0, 0)),
            pl.BlockSpec((1, D), lambda i: (0, 0)),
        ],
        out_specs=pl.BlockSpec((blk, D), lambda i: (i, 0)),
        out_shape=jax.ShapeDtypeStruct((N, D), jnp.float32),
    )(x, gamma.reshape(1, D), beta.reshape(1, D))


_sc_mesh = plsc.VectorSubcoreMesh(core_axis_name="c", subcore_axis_name="s")

_sc_params = pltpu.CompilerParams(
    needs_layout_passes=False,
    use_tc_tiling_on_sc=False,
)


@functools.partial(
    pl.kernel,
    out_type=(
        jax.ShapeDtypeStruct((NC, N_PAD, D), jnp.float32),
        jax.ShapeDtypeStruct((NC, CROWS, L), jnp.float32),
    ),
    mesh=_sc_mesh,
    compiler_params=_sc_params,
    scratch_types=[
        pltpu.VMEM((CROWS, L), jnp.float32),
        pltpu.VMEM((5, 128), jnp.int32),
        pltpu.VMEM_SHARED((N_PAD, D), jnp.float32),
        pltpu.VMEM_SHARED((CROWS, L), jnp.float32),
    ] + [pltpu.VMEM((CH, D), jnp.float32)] * NBUF
      + [pltpu.VMEM((CH,), jnp.int32)] * NBUF
      + [pltpu.VMEM((CH,), jnp.int32)] * NBUF
      + [pltpu.SemaphoreType.DMA] * (4 * NBUF),
)
def _sc_aggregate(h_hbm, src_hbm, dst_hbm, aggr_out, cnt_out,
                  cnt_l, rowid_v,
                  aggr_sh, cnt_sh, *rest):
    rows = rest[:NBUF]
    srcs = rest[NBUF:2 * NBUF]
    dsts = rest[2 * NBUF:3 * NBUF]
    gsem = rest[3 * NBUF:4 * NBUF]
    ssem = rest[4 * NBUF:5 * NBUF]
    isem = rest[5 * NBUF:6 * NBUF]
    dsem = rest[6 * NBUF:]
    c = lax.axis_index("c")
    s = lax.axis_index("s")

    @pl.loop(0, CH)
    def _(i):
        @pl.loop(0, D, step=L)
        def _(j):
            rows[0][i, pl.ds(j, L)] = jnp.zeros((L,), jnp.float32)

    @pl.loop(0, CROWS)
    def _(i):
        cnt_l[i, pl.ds(0, L)] = jnp.zeros((L,), jnp.float32)

    for k in range(5):
        for j in range(8):
            rowid_v[k, pl.ds(j * L, L)] = (
                lax.iota(jnp.int32, L) + (k * 128 + j * L)
            )

    for k in range(0, RPT, CH):
        nb = min(CH, RPT - k)
        pltpu.sync_copy(rows[0].at[pl.ds(0, nb)],
                        aggr_sh.at[pl.ds(s * RPT + k, nb)])

    @pl.when(s == 0)
    def _():
        for k in range(0, CROWS, 128):
            pltpu.sync_copy(cnt_l.at[pl.ds(k, 128)],
                            cnt_sh.at[pl.ds(k, 128)])

    plsc.subcore_barrier()

    wid = c * NS + s
    e0 = wid * EPT

    def src_desc(i, b):
        return pltpu.make_async_copy(
            src_hbm.at[pl.ds(e0 + i * CH, CH)], srcs[b], isem[b])

    def dst_desc(i, b):
        return pltpu.make_async_copy(
            dst_hbm.at[pl.ds(e0 + i * CH, CH)], dsts[b], dsem[b])

    def gather_desc(b):
        return pltpu.make_async_copy(
            h_hbm.at[srcs[b].at[pl.ds(0, CH)]], rows[b], gsem[b])

    def scatter_desc(b):
        return pltpu.make_async_copy(
            rows[b], aggr_sh.at[dsts[b].at[pl.ds(0, CH)]], ssem[b])

    def count_update(b):
        for j in range(CH // L):
            dv = dsts[b][pl.ds(j * L, L)]
            row = lax.shift_right_logical(dv, 4)
            col = lax.bitwise_and(dv, 15)
            plsc.addupdate_scatter(cnt_l, [row, col],
                                   jnp.full((L,), 1.0, jnp.float32))

    for b in range(NBUF):
        src_desc(b, b).start()
        dst_desc(b, b).start()
    for b in range(NBUF):
        src_desc(b, b).wait()
        gather_desc(b).start()

    @pl.loop(0, NGROUP)
    def _(g):
        for b in range(NBUF):
            i = g * NBUF + b
            gather_desc(b).wait()

            @pl.when(g < NGROUP - 1)
            def _():
                src_desc(i + NBUF, b).start()

            dst_desc(i, b).wait()
            scatter_desc(b).start(add=True)
            count_update(b)
        for b in range(NBUF):
            i = g * NBUF + b
            scatter_desc(b).wait()

            @pl.when(g < NGROUP - 1)
            def _():
                dst_desc(i + NBUF, b).start()
                src_desc(i + NBUF, b).wait()
                gather_desc(b).start()

    for i in range(NRING, NCHUNK):
        pltpu.sync_copy(src_hbm.at[pl.ds(e0 + i * CH, CH)], srcs[0])
        pltpu.sync_copy(dst_hbm.at[pl.ds(e0 + i * CH, CH)], dsts[0])
        pltpu.sync_copy(h_hbm.at[srcs[0].at[pl.ds(0, CH)]], rows[0])
        pltpu.sync_copy(rows[0], aggr_sh.at[dsts[0].at[pl.ds(0, CH)]], add=True)
        count_update(0)

    plsc.subcore_barrier()

    for k in range(5):
        pltpu.sync_copy(cnt_l.at[pl.ds(k * 128, 128)],
                        cnt_sh.at[rowid_v.at[k]], add=True)

    plsc.subcore_barrier()

    for k, nb in RBLOCKS:
        r0 = s * RPT + k
        pltpu.sync_copy(aggr_sh.at[pl.ds(r0, nb)],
                        aggr_out.at[c, pl.ds(r0, nb)])

    @pl.when(s == 0)
    def _():
        pltpu.sync_copy(cnt_sh, cnt_out.at[c])


def _combine_body(a0_ref, a1_ref, c0_ref, c1_ref, h_ref, wl_ref, bl_ref,
                  wr_ref, o_ref):
    aggr = a0_ref[...] + a1_ref[...]
    cnt = jnp.maximum(c0_ref[...] + c1_ref[...], 1.0)
    mean = aggr / cnt
    dims = (((1,), (1,)), ((), ()))
    o_ref[...] = (
        lax.dot_general(mean, wl_ref[...], dims,
                        preferred_element_type=jnp.float32)
        + bl_ref[...]
        + lax.dot_general(h_ref[...], wr_ref[...], dims,
                          preferred_element_type=jnp.float32)
    )


def _combine(a0, a1, c0, c1, h, W_l, b_l, W_r):
    blk = 1000
    return pl.pallas_call(
        _combine_body,
        grid=(N // blk,),
        in_specs=[
            pl.BlockSpec((blk, D), lambda i: (i, 0)),
            pl.BlockSpec((blk, D), lambda i: (i, 0)),
            pl.BlockSpec((blk, 1), lambda i: (i, 0)),
            pl.BlockSpec((blk, 1), lambda i: (i, 0)),
            pl.BlockSpec((blk, D), lambda i: (i, 0)),
            pl.BlockSpec((D, D), lambda i: (0, 0)),
            pl.BlockSpec((1, D), lambda i: (0, 0)),
            pl.BlockSpec((D, D), lambda i: (0, 0)),
        ],
        out_specs=pl.BlockSpec((blk, D), lambda i: (i, 0)),
        out_shape=jax.ShapeDtypeStruct((N, D), jnp.float32),
    )(a0, a1, c0, c1, h, W_l, b_l.reshape(1, D), W_r)


def kernel(x, edge_index, ln_gamma, ln_beta, W_l, b_l, W_r):
    ei = edge_index.astype(jnp.int32)
    src = ei[0]
    dst = ei[1]
    h = _ln_relu(x, ln_gamma, ln_beta)
    aggr_p, cnt_p = _sc_aggregate(h, src, dst)
    aggr_p = aggr_p[:, :N, :]
    cnt = cnt_p.reshape(NC, CROWS * L)[:, :N]
    out = _combine(aggr_p[0], aggr_p[1],
                   cnt[0].reshape(N, 1), cnt[1].reshape(N, 1),
                   h, W_l, b_l, W_r)
    return out


# speedup vs baseline: 12.0728x; 1.0052x over previous
import dataclasses
import functools

import jax
import jax.numpy as jnp
from jax import lax
from jax.experimental import pallas as pl
from jax.experimental.pallas import tpu as pltpu
from jax.experimental.pallas import tpu_sc as plsc

N = 10000
D = 128
E = 320000

NC = 2
NS = 16
L = 16
EPC = E // NC
EPT = EPC // NS
CH = 80
NCHUNK = EPT // CH
NBUF = 3
NGROUP = NCHUNK // NBUF
NRING = NGROUP * NBUF
N_PAD = 10112
RPT = N_PAD // NS
RBLOCKS = [(k, min(128, RPT - k)) for k in range(0, RPT, 128)]
CROWS = 640


def _ln_relu_body(x_ref, g_ref, b_ref, h_ref):
    x = x_ref[...]
    m = jnp.mean(x, axis=1, keepdims=True)
    d = x - m
    v = jnp.mean(d * d, axis=1, keepdims=True)
    h = d * lax.rsqrt(v + 1e-5) * g_ref[...] + b_ref[...]
    h_ref[...] = jnp.maximum(h, 0.0)


def _ln_relu(x, gamma, beta):
    blk = 1000
    return pl.pallas_call(
        _ln_relu_body,
        grid=(N // blk,),
        in_specs=[
            pl.BlockSpec((blk, D), lambda i: (i, 0)),
            pl.BlockSpec((1, D), lambda i: (0, 0)),
            pl.BlockSpec((1, D), lambda i: (0, 0)),
        ],
        out_specs=pl.BlockSpec((blk, D), lambda i: (i, 0)),
        out_shape=jax.ShapeDtypeStruct((N, D), jnp.float32),
    )(x, gamma.reshape(1, D), beta.reshape(1, D))


_sc_mesh = plsc.VectorSubcoreMesh(core_axis_name="c", subcore_axis_name="s")

_sc_params = pltpu.CompilerParams(
    needs_layout_passes=False,
    use_tc_tiling_on_sc=False,
)


@functools.partial(
    pl.kernel,
    out_type=(
        jax.ShapeDtypeStruct((NC, N_PAD, D), jnp.float32),
        jax.ShapeDtypeStruct((NC, CROWS, L), jnp.float32),
    ),
    mesh=_sc_mesh,
    compiler_params=_sc_params,
    scratch_types=[
        pltpu.VMEM((CROWS, L), jnp.float32),
        pltpu.VMEM((5, 128), jnp.int32),
        pltpu.VMEM_SHARED((N_PAD, D), jnp.float32),
        pltpu.VMEM_SHARED((CROWS, L), jnp.float32),
    ] + [pltpu.VMEM((CH, D), jnp.float32)] * NBUF
      + [pltpu.VMEM((CH,), jnp.int32)] * NBUF
      + [pltpu.VMEM((CH,), jnp.int32)] * NBUF
      + [pltpu.SemaphoreType.DMA] * (4 * NBUF),
)
def _sc_aggregate(h_hbm, src_hbm, dst_hbm, aggr_out, cnt_out,
                  cnt_l, rowid_v,
                  aggr_sh, cnt_sh, *rest):
    rows = rest[:NBUF]
    srcs = rest[NBUF:2 * NBUF]
    dsts = rest[2 * NBUF:3 * NBUF]
    gsem = rest[3 * NBUF:4 * NBUF]
    ssem = rest[4 * NBUF:5 * NBUF]
    isem = rest[5 * NBUF:6 * NBUF]
    dsem = rest[6 * NBUF:]
    c = lax.axis_index("c")
    s = lax.axis_index("s")

    @pl.loop(0, CH)
    def _(i):
        @pl.loop(0, D, step=L)
        def _(j):
            rows[0][i, pl.ds(j, L)] = jnp.zeros((L,), jnp.float32)

    @pl.loop(0, CROWS)
    def _(i):
        cnt_l[i, pl.ds(0, L)] = jnp.zeros((L,), jnp.float32)

    for k in range(5):
        for j in range(8):
            rowid_v[k, pl.ds(j * L, L)] = (
                lax.iota(jnp.int32, L) + (k * 128 + j * L)
            )

    sems = rest[3 * NBUF:]
    zdescs = []
    for z, k in enumerate(range(0, RPT, CH)):
        nb = min(CH, RPT - k)
        d = pltpu.make_async_copy(rows[0].at[pl.ds(0, nb)],
                                  aggr_sh.at[pl.ds(s * RPT + k, nb)],
                                  sems[z])
        d.start()
        zdescs.append(d)
    for d in zdescs:
        d.wait()

    @pl.when(s == 0)
    def _():
        zc = []
        for z, k in enumerate(range(0, CROWS, 128)):
            d = pltpu.make_async_copy(cnt_l.at[pl.ds(k, 128)],
                                      cnt_sh.at[pl.ds(k, 128)], sems[z])
            d.start()
            zc.append(d)
        for d in zc:
            d.wait()

    plsc.subcore_barrier()

    wid = c * NS + s
    e0 = wid * EPT

    def src_desc(i, b):
        return pltpu.make_async_copy(
            src_hbm.at[pl.ds(e0 + i * CH, CH)], srcs[b], isem[b])

    def dst_desc(i, b):
        return pltpu.make_async_copy(
            dst_hbm.at[pl.ds(e0 + i * CH, CH)], dsts[b], dsem[b])

    def gather_desc(b):
        return pltpu.make_async_copy(
            h_hbm.at[srcs[b].at[pl.ds(0, CH)]], rows[b], gsem[b])

    def scatter_desc(b):
        return pltpu.make_async_copy(
            rows[b], aggr_sh.at[dsts[b].at[pl.ds(0, CH)]], ssem[b])

    def count_update(b):
        for j in range(CH // L):
            dv = dsts[b][pl.ds(j * L, L)]
            row = lax.shift_right_logical(dv, 4)
            col = lax.bitwise_and(dv, 15)
            plsc.addupdate_scatter(cnt_l, [row, col],
                                   jnp.full((L,), 1.0, jnp.float32))

    for b in range(NBUF):
        src_desc(b, b).start()
        dst_desc(b, b).start()
    for b in range(NBUF):
        src_desc(b, b).wait()
        gather_desc(b).start()

    @pl.loop(0, NGROUP)
    def _(g):
        for b in range(NBUF):
            i = g * NBUF + b
            gather_desc(b).wait()

            @pl.when(g < NGROUP - 1)
            def _():
                src_desc(i + NBUF, b).start()

            dst_desc(i, b).wait()
            scatter_desc(b).start(add=True)
            count_update(b)
        for b in range(NBUF):
            i = g * NBUF + b
            scatter_desc(b).wait()

            @pl.when(g < NGROUP - 1)
            def _():
                dst_desc(i + NBUF, b).start()
                src_desc(i + NBUF, b).wait()
                gather_desc(b).start()

    for i in range(NRING, NCHUNK):
        pltpu.sync_copy(src_hbm.at[pl.ds(e0 + i * CH, CH)], srcs[0])
        pltpu.sync_copy(dst_hbm.at[pl.ds(e0 + i * CH, CH)], dsts[0])
        pltpu.sync_copy(h_hbm.at[srcs[0].at[pl.ds(0, CH)]], rows[0])
        pltpu.sync_copy(rows[0], aggr_sh.at[dsts[0].at[pl.ds(0, CH)]], add=True)
        count_update(0)

    plsc.subcore_barrier()

    mdescs = []
    for k in range(5):
        d = pltpu.make_async_copy(cnt_l.at[pl.ds(k * 128, 128)],
                                  cnt_sh.at[rowid_v.at[k]], sems[k])
        d.start(add=True)
        mdescs.append(d)
    for d in mdescs:
        d.wait()

    plsc.subcore_barrier()

    wdescs = []
    for z, (k, nb) in enumerate(RBLOCKS):
        r0 = s * RPT + k
        d = pltpu.make_async_copy(aggr_sh.at[pl.ds(r0, nb)],
                                  aggr_out.at[c, pl.ds(r0, nb)], sems[z])
        d.start()
        wdescs.append(d)

    @pl.when(s == 0)
    def _():
        d = pltpu.make_async_copy(cnt_sh, cnt_out.at[c], sems[len(RBLOCKS)])
        d.start()
        d.wait()

    for d in wdescs:
        d.wait()


def _combine_body(a0_ref, a1_ref, c0_ref, c1_ref, h_ref, wl_ref, bl_ref,
                  wr_ref, o_ref):
    aggr = a0_ref[...] + a1_ref[...]
    cnt = jnp.maximum(c0_ref[...] + c1_ref[...], 1.0)
    mean = aggr / cnt
    dims = (((1,), (1,)), ((), ()))
    o_ref[...] = (
        lax.dot_general(mean, wl_ref[...], dims,
                        preferred_element_type=jnp.float32)
        + bl_ref[...]
        + lax.dot_general(h_ref[...], wr_ref[...], dims,
                          preferred_element_type=jnp.float32)
    )


def _combine(a0, a1, c0, c1, h, W_l, b_l, W_r):
    blk = 1000
    return pl.pallas_call(
        _combine_body,
        grid=(N // blk,),
        in_specs=[
            pl.BlockSpec((blk, D), lambda i: (i, 0)),
            pl.BlockSpec((blk, D), lambda i: (i, 0)),
            pl.BlockSpec((blk, 1), lambda i: (i, 0)),
            pl.BlockSpec((blk, 1), lambda i: (i, 0)),
            pl.BlockSpec((blk, D), lambda i: (i, 0)),
            pl.BlockSpec((D, D), lambda i: (0, 0)),
            pl.BlockSpec((1, D), lambda i: (0, 0)),
            pl.BlockSpec((D, D), lambda i: (0, 0)),
        ],
        out_specs=pl.BlockSpec((blk, D), lambda i: (i, 0)),
        out_shape=jax.ShapeDtypeStruct((N, D), jnp.float32),
    )(a0, a1, c0, c1, h, W_l, b_l.reshape(1, D), W_r)


def kernel(x, edge_index, ln_gamma, ln_beta, W_l, b_l, W_r):
    ei = edge_index.astype(jnp.int32)
    src = ei[0]
    dst = ei[1]
    h = _ln_relu(x, ln_gamma, ln_beta)
    aggr_p, cnt_p = _sc_aggregate(h, src, dst)
    aggr_p = aggr_p[:, :N, :]
    cnt = cnt_p.reshape(NC, CROWS * L)[:, :N]
    out = _combine(aggr_p[0], aggr_p[1],
                   cnt[0].reshape(N, 1), cnt[1].reshape(N, 1),
                   h, W_l, b_l, W_r)
    return out
